# baseline (speedup 1.0000x reference)
import jax
import jax.numpy as jnp
from jax import lax
from jax.experimental import pallas as pl
from jax.experimental.pallas import tpu as pltpu
from jax.experimental.pallas import tpu_sc as plsc

N = 10000
E = 320000
D_IN = 128
HID = 64
OUT = 64
B = 16

DF = 64
DD = 16
NC = 2
NS = 16
NW = NC * NS
EP = E // NW
CK = 80
NCH = EP // CK
NT = 10
RPT = N // NT
RC = 200
BR = 400
GRID = N // BR

_EPS = 1e-16


def _transform_body(h_ref, w_ref, asrc_ref, adst_ref, xp_ref, as_ref, ad_ref):
    xp = jnp.dot(h_ref[...], w_ref[...], preferred_element_type=jnp.float32)
    as_ref[...] = jnp.sum(xp * asrc_ref[...], axis=1, keepdims=True)
    ad_ref[...] = jnp.sum(xp * adst_ref[...], axis=1, keepdims=True)
    xp_ref[...] = xp


def _transform(h, w, a_src, a_dst):
    d = h.shape[1]
    return pl.pallas_call(
        _transform_body,
        grid=(GRID,),
        in_specs=[
            pl.BlockSpec((BR, d), lambda i: (i, 0)),
            pl.BlockSpec((d, 64), lambda i: (0, 0)),
            pl.BlockSpec((1, 64), lambda i: (0, 0)),
            pl.BlockSpec((1, 64), lambda i: (0, 0)),
        ],
        out_specs=[
            pl.BlockSpec((BR, DF), lambda i: (i, 0)),
            pl.BlockSpec((BR, 1), lambda i: (i, 0)),
            pl.BlockSpec((BR, 1), lambda i: (i, 0)),
        ],
        out_shape=[
            jax.ShapeDtypeStruct((N, DF), jnp.float32),
            jax.ShapeDtypeStruct((N, 1), jnp.float32),
            jax.ShapeDtypeStruct((N, 1), jnp.float32),
        ],
    )(h, w, a_src, a_dst)


def _mid_body(p64_ref, p16_ref, b_ref, w_ref, asrc_ref, adst_ref,
              xp_ref, as_ref, ad_ref):
    den = (p16_ref[0] + p16_ref[1])[:, 0:1]
    h = (p64_ref[0] + p64_ref[1]) / (den + _EPS) + b_ref[...]
    h = jnp.where(h > 0, h, jnp.exp(jnp.minimum(h, 0.0)) - 1.0)
    xp = jnp.dot(h, w_ref[...], preferred_element_type=jnp.float32)
    as_ref[...] = jnp.sum(xp * asrc_ref[...], axis=1, keepdims=True)
    ad_ref[...] = jnp.sum(xp * adst_ref[...], axis=1, keepdims=True)
    xp_ref[...] = xp


def _mid(p64, p16, b, w, a_src, a_dst):
    return pl.pallas_call(
        _mid_body,
        grid=(GRID,),
        in_specs=[
            pl.BlockSpec((2, BR, DF), lambda i: (0, i, 0)),
            pl.BlockSpec((2, BR, DD), lambda i: (0, i, 0)),
            pl.BlockSpec((1, 64), lambda i: (0, 0)),
            pl.BlockSpec((64, 64), lambda i: (0, 0)),
            pl.BlockSpec((1, 64), lambda i: (0, 0)),
            pl.BlockSpec((1, 64), lambda i: (0, 0)),
        ],
        out_specs=[
            pl.BlockSpec((BR, DF), lambda i: (i, 0)),
            pl.BlockSpec((BR, 1), lambda i: (i, 0)),
            pl.BlockSpec((BR, 1), lambda i: (i, 0)),
        ],
        out_shape=[
            jax.ShapeDtypeStruct((N, DF), jnp.float32),
            jax.ShapeDtypeStruct((N, 1), jnp.float32),
            jax.ShapeDtypeStruct((N, 1), jnp.float32),
        ],
    )(p64, p16, b, w, a_src, a_dst)


def _pool_body(p64_ref, p16_ref, b_ref, batch_ref, out_ref, cnt_ref):
    i = pl.program_id(0)

    @pl.when(i == 0)
    def _():
        out_ref[...] = jnp.zeros_like(out_ref)
        cnt_ref[...] = jnp.zeros_like(cnt_ref)

    den = (p16_ref[0] + p16_ref[1])[:, 0:1]
    h = (p64_ref[0] + p64_ref[1]) / (den + _EPS) + b_ref[...]
    seg = lax.broadcasted_iota(jnp.int32, (BR, B), 1).astype(jnp.float32)
    oh = jnp.where(batch_ref[...] == seg, 1.0, 0.0).astype(jnp.float32)
    out_ref[...] += lax.dot_general(
        oh, h, (((0,), (0,)), ((), ())), preferred_element_type=jnp.float32)
    cnt_ref[...] += jnp.broadcast_to(
        jnp.sum(oh, axis=0, keepdims=True).T, (B, 64))

    @pl.when(i == GRID - 1)
    def _():
        out_ref[...] = out_ref[...] / jnp.maximum(cnt_ref[...], 1.0)


def _pool(p64, p16, b, batchf):
    return pl.pallas_call(
        _pool_body,
        grid=(GRID,),
        in_specs=[
            pl.BlockSpec((2, BR, DF), lambda i: (0, i, 0)),
            pl.BlockSpec((2, BR, DD), lambda i: (0, i, 0)),
            pl.BlockSpec((1, 64), lambda i: (0, 0)),
            pl.BlockSpec((BR, 1), lambda i: (i, 0)),
        ],
        out_specs=pl.BlockSpec((B, 64), lambda i: (0, 0)),
        out_shape=jax.ShapeDtypeStruct((B, 64), jnp.float32),
        scratch_shapes=[pltpu.VMEM((B, 64), jnp.float32)],
    )(p64, p16, b, batchf)


def _edge_body(src_h, dst_h, as_h, ad_h, xp_h, part64_h, part16_h,
               src_t, dst_t, as_t, ad_t,
               b_0, b_1, b_2, b_3, d_0, d_1, stage, dstage, acc, dacc,
               gsem0, gsem1, gsem2, gsem3,
               ssem0, ssem1, ssem2, ssem3, dsem0, dsem1):
    buf = [b_0, b_1, b_2, b_3]
    dbuf = [d_0, d_1]
    gsem = [gsem0, gsem1, gsem2, gsem3]
    ssem = [ssem0, ssem1, ssem2, ssem3]
    dsem = [dsem0, dsem1]
    c = lax.axis_index("c")
    s = lax.axis_index("s")
    g = s * NC + c

    def _zrow(r, _):
        for j in range(DF // 16):
            stage[r, pl.ds(j * 16, 16)] = jnp.zeros((16,), jnp.float32)
        dstage[r, :] = jnp.zeros((16,), jnp.float32)
        return 0
    lax.fori_loop(0, RC, _zrow, 0)
    base = s * RPT

    @pl.when(s < NT)
    def _():
        for j in range(RPT // RC):
            pltpu.sync_copy(stage, acc.at[pl.ds(base + j * RC, RC)])
            pltpu.sync_copy(dstage, dacc.at[pl.ds(base + j * RC, RC)])

    pltpu.sync_copy(src_h.at[g], src_t)
    pltpu.sync_copy(dst_h.at[g], dst_t)
    pltpu.sync_copy(as_h, as_t)
    pltpu.sync_copy(ad_h, ad_t)

    plsc.subcore_barrier()

    def _gather(ch, p):
        pltpu.async_copy(xp_h.at[src_t.at[ch]], buf[p], gsem[p])

    def _gwait(ch, p):
        pltpu.make_async_copy(xp_h.at[src_t.at[ch]], buf[p], gsem[p]).wait()

    def _scat(ch, p):
        pltpu.async_copy(buf[p], acc.at[dst_t.at[ch]], ssem[p], add=True)

    def _swait(ch, p):
        pltpu.make_async_copy(buf[p], acc.at[dst_t.at[ch]], ssem[p]).wait()

    def _dscat(ch, p2):
        pltpu.async_copy(dbuf[p2], dacc.at[dst_t.at[ch]], dsem[p2], add=True)

    def _dwait(ch, p2):
        pltpu.make_async_copy(
            dbuf[p2], dacc.at[dst_t.at[ch]], dsem[p2]).wait()

    def _scale(ch, p, p2):
        for jj in range(CK // 16):
            s16 = src_t[ch, pl.ds(jj * 16, 16)]
            d16 = dst_t[ch, pl.ds(jj * 16, 16)]
            e = plsc.load_gather(as_t, [s16]) + plsc.load_gather(ad_t, [d16])
            e = jnp.where(e >= 0, e, 0.2 * e)
            ee = jnp.exp(e)
            for ii in range(16):
                eei = ee[ii]
                r = jj * 16 + ii
                for j in range(DF // 16):
                    buf[p][r, pl.ds(j * 16, 16)] = (
                        buf[p][r, pl.ds(j * 16, 16)] * eei)
                dbuf[p2][r, :] = jnp.broadcast_to(eei, (16,))

    def _chunk(ch, p, p2):
        q = (p + 2) % 4

        @pl.when(ch >= 2)
        def _():
            _swait(ch - 2, q)
            _dwait(ch - 2, p2)

        @pl.when(ch + 2 < NCH)
        def _():
            _gather(ch + 2, q)

        _gwait(ch, p)
        _scale(ch, p, p2)
        _scat(ch, p)
        _dscat(ch, p2)

    _gather(0, 0)
    _gather(1, 1)

    def _quad(k, _):
        ch0 = 4 * k
        _chunk(ch0, 0, 0)
        _chunk(ch0 + 1, 1, 1)
        _chunk(ch0 + 2, 2, 0)
        _chunk(ch0 + 3, 3, 1)
        return 0

    lax.fori_loop(0, (NCH - 1) // 4, _quad, 0)
    _chunk(jnp.int32(NCH - 1), 0, 0)
    _swait(jnp.int32(NCH - 2), 3)
    _swait(jnp.int32(NCH - 1), 0)
    _dwait(jnp.int32(NCH - 2), 1)
    _dwait(jnp.int32(NCH - 1), 0)

    plsc.subcore_barrier()

    @pl.when(s < NT)
    def _():
        for j in range(RPT // RC):
            pltpu.sync_copy(acc.at[pl.ds(base + j * RC, RC)], stage)
            pltpu.sync_copy(stage, part64_h.at[c, pl.ds(base + j * RC, RC)])
            pltpu.sync_copy(dacc.at[pl.ds(base + j * RC, RC)], dstage)
            pltpu.sync_copy(dstage, part16_h.at[c, pl.ds(base + j * RC, RC)])


def _edge_phase(src3, dst3, alpha_s, alpha_d, xp):
    mesh = plsc.VectorSubcoreMesh(
        core_axis_name="c", subcore_axis_name="s",
        num_cores=NC, num_subcores=NS)
    k = pl.kernel(
        _edge_body,
        out_type=[
            jax.ShapeDtypeStruct((NC, N, DF), jnp.float32),
            jax.ShapeDtypeStruct((NC, N, DD), jnp.float32),
        ],
        mesh=mesh,
        compiler_params=pltpu.CompilerParams(
            needs_layout_passes=False, use_tc_tiling_on_sc=False),
        scratch_types=[
            pltpu.VMEM((NCH, CK), jnp.int32),
            pltpu.VMEM((NCH, CK), jnp.int32),
            pltpu.VMEM((N,), jnp.float32),
            pltpu.VMEM((N,), jnp.float32),
            pltpu.VMEM((CK, DF), jnp.float32),
            pltpu.VMEM((CK, DF), jnp.float32),
            pltpu.VMEM((CK, DF), jnp.float32),
            pltpu.VMEM((CK, DF), jnp.float32),
            pltpu.VMEM((CK, DD), jnp.float32),
            pltpu.VMEM((CK, DD), jnp.float32),
            pltpu.VMEM((RC, DF), jnp.float32),
            pltpu.VMEM((RC, DD), jnp.float32),
            pltpu.VMEM_SHARED((N, DF), jnp.float32),
            pltpu.VMEM_SHARED((N, DD), jnp.float32),
        ] + [pltpu.SemaphoreType.DMA] * 10,
    )
    return k(src3, dst3, alpha_s, alpha_d, xp)


@jax.jit
def kernel(x, edge_index, batch, W0, a_src0, a_dst0, b0,
           W1, a_src1, a_dst1, b1):
    src3 = edge_index[0].reshape(NW, NCH, CK)
    dst3 = edge_index[1].reshape(NW, NCH, CK)
    batchf = batch.astype(jnp.float32).reshape(N, 1)

    xp0, as0, ad0 = _transform(x, W0, a_src0, a_dst0)
    p64_0, p16_0 = _edge_phase(src3, dst3, as0.reshape(N), ad0.reshape(N),
                               xp0)
    xp1, as1, ad1 = _mid(p64_0, p16_0, b0.reshape(1, 64), W1, a_src1, a_dst1)
    p64_1, p16_1 = _edge_phase(src3, dst3, as1.reshape(N), ad1.reshape(N),
                               xp1)
    return _pool(p64_1, p16_1, b1.reshape(1, 64), batchf)


# speedup vs baseline: 1.0898x; 1.0898x over previous
import functools

import jax
import jax.numpy as jnp
from jax import lax
from jax.experimental import pallas as pl
from jax.experimental.pallas import tpu as pltpu
from jax.experimental.pallas import tpu_sc as plsc

N = 10000
E = 320000
D_IN = 128
HID = 64
OUT = 64
B = 16

DE = 80
NC = 2
NS = 16
NW = NC * NS
EP = E // NW
CK = 80
NCH = EP // CK
NBUF = 5
NT = 16
RPT = N // NT
RC = 125
BR = 400
GRID = N // BR

_EPS = 1e-16


def _transform_body(h_ref, w_ref, asrc_ref, adst_ref, xpe_ref, as_ref, ad_ref):
    xp = jnp.dot(h_ref[...], w_ref[...], preferred_element_type=jnp.float32)
    as_ref[...] = jnp.sum(xp * asrc_ref[...], axis=1, keepdims=True)
    ad_ref[...] = jnp.sum(xp * adst_ref[...], axis=1, keepdims=True)
    lane = lax.broadcasted_iota(jnp.int32, (BR, DE - 64), 1)
    pad = jnp.where(lane == 0, 1.0, 0.0).astype(jnp.float32)
    xpe_ref[...] = jnp.concatenate([xp, pad], axis=1)


def _transform(h, w, a_src, a_dst):
    d = h.shape[1]
    return pl.pallas_call(
        _transform_body,
        grid=(GRID,),
        in_specs=[
            pl.BlockSpec((BR, d), lambda i: (i, 0)),
            pl.BlockSpec((d, 64), lambda i: (0, 0)),
            pl.BlockSpec((1, 64), lambda i: (0, 0)),
            pl.BlockSpec((1, 64), lambda i: (0, 0)),
        ],
        out_specs=[
            pl.BlockSpec((BR, DE), lambda i: (i, 0)),
            pl.BlockSpec((BR, 1), lambda i: (i, 0)),
            pl.BlockSpec((BR, 1), lambda i: (i, 0)),
        ],
        out_shape=[
            jax.ShapeDtypeStruct((N, DE), jnp.float32),
            jax.ShapeDtypeStruct((N, 1), jnp.float32),
            jax.ShapeDtypeStruct((N, 1), jnp.float32),
        ],
    )(h, w, a_src, a_dst)


def _mid_body(p_ref, b_ref, w_ref, asrc_ref, adst_ref,
              xpe_ref, as_ref, ad_ref):
    ps = p_ref[0] + p_ref[1]
    den = ps[:, 64:65]
    h = ps[:, :64] / (den + _EPS) + b_ref[...]
    h = jnp.where(h > 0, h, jnp.exp(jnp.minimum(h, 0.0)) - 1.0)
    xp = jnp.dot(h, w_ref[...], preferred_element_type=jnp.float32)
    as_ref[...] = jnp.sum(xp * asrc_ref[...], axis=1, keepdims=True)
    ad_ref[...] = jnp.sum(xp * adst_ref[...], axis=1, keepdims=True)
    lane = lax.broadcasted_iota(jnp.int32, (BR, DE - 64), 1)
    pad = jnp.where(lane == 0, 1.0, 0.0).astype(jnp.float32)
    xpe_ref[...] = jnp.concatenate([xp, pad], axis=1)


def _mid(p, b, w, a_src, a_dst):
    return pl.pallas_call(
        _mid_body,
        grid=(GRID,),
        in_specs=[
            pl.BlockSpec((2, BR, DE), lambda i: (0, i, 0)),
            pl.BlockSpec((1, 64), lambda i: (0, 0)),
            pl.BlockSpec((64, 64), lambda i: (0, 0)),
            pl.BlockSpec((1, 64), lambda i: (0, 0)),
            pl.BlockSpec((1, 64), lambda i: (0, 0)),
        ],
        out_specs=[
            pl.BlockSpec((BR, DE), lambda i: (i, 0)),
            pl.BlockSpec((BR, 1), lambda i: (i, 0)),
            pl.BlockSpec((BR, 1), lambda i: (i, 0)),
        ],
        out_shape=[
            jax.ShapeDtypeStruct((N, DE), jnp.float32),
            jax.ShapeDtypeStruct((N, 1), jnp.float32),
            jax.ShapeDtypeStruct((N, 1), jnp.float32),
        ],
    )(p, b, w, a_src, a_dst)


def _pool_body(p_ref, b_ref, batch_ref, out_ref, cnt_ref):
    i = pl.program_id(0)

    @pl.when(i == 0)
    def _():
        out_ref[...] = jnp.zeros_like(out_ref)
        cnt_ref[...] = jnp.zeros_like(cnt_ref)

    ps = p_ref[0] + p_ref[1]
    den = ps[:, 64:65]
    h = ps[:, :64] / (den + _EPS) + b_ref[...]
    seg = lax.broadcasted_iota(jnp.int32, (BR, B), 1).astype(jnp.float32)
    oh = jnp.where(batch_ref[...] == seg, 1.0, 0.0).astype(jnp.float32)
    out_ref[...] += lax.dot_general(
        oh, h, (((0,), (0,)), ((), ())), preferred_element_type=jnp.float32)
    cnt_ref[...] += jnp.broadcast_to(
        jnp.sum(oh, axis=0, keepdims=True).T, (B, 64))

    @pl.when(i == GRID - 1)
    def _():
        out_ref[...] = out_ref[...] / jnp.maximum(cnt_ref[...], 1.0)


def _pool(p, b, batchf):
    return pl.pallas_call(
        _pool_body,
        grid=(GRID,),
        in_specs=[
            pl.BlockSpec((2, BR, DE), lambda i: (0, i, 0)),
            pl.BlockSpec((1, 64), lambda i: (0, 0)),
            pl.BlockSpec((BR, 1), lambda i: (i, 0)),
        ],
        out_specs=pl.BlockSpec((B, 64), lambda i: (0, 0)),
        out_shape=jax.ShapeDtypeStruct((B, 64), jnp.float32),
        scratch_shapes=[pltpu.VMEM((B, 64), jnp.float32)],
    )(p, b, batchf)


def _edge_body(src_h, dst_h, as_h, ad_h, xpe_h, part_h,
               src_t, dst_t, as_t, ad_t,
               b_0, b_1, b_2, stage, acc,
               gsem0, gsem1, gsem2, ssem0, ssem1, ssem2):
    buf = [b_0, b_1, b_2]
    gsem = [gsem0, gsem1, gsem2]
    ssem = [ssem0, ssem1, ssem2]
    c = lax.axis_index("c")
    s = lax.axis_index("s")
    g = s * NC + c

    def _zrow(r, _):
        for j in range(DE // 16):
            stage[r, pl.ds(j * 16, 16)] = jnp.zeros((16,), jnp.float32)
        return 0
    lax.fori_loop(0, RC, _zrow, 0)
    base = s * RPT

    @pl.when(s < NT)
    def _():
        for j in range(RPT // RC):
            pltpu.sync_copy(stage, acc.at[pl.ds(base + j * RC, RC)])

    pltpu.sync_copy(src_h.at[g], src_t)
    pltpu.sync_copy(dst_h.at[g], dst_t)
    pltpu.sync_copy(as_h, as_t)
    pltpu.sync_copy(ad_h, ad_t)

    plsc.subcore_barrier()

    def _gather(ch, p):
        pltpu.async_copy(xpe_h.at[src_t.at[ch]], buf[p], gsem[p])

    def _gwait(ch, p):
        pltpu.make_async_copy(xpe_h.at[src_t.at[ch]], buf[p], gsem[p]).wait()

    def _scat(ch, p):
        pltpu.async_copy(buf[p], acc.at[dst_t.at[ch]], ssem[p], add=True)

    def _swait(ch, p):
        pltpu.make_async_copy(buf[p], acc.at[dst_t.at[ch]], ssem[p]).wait()

    def _scale(ch, p):
        for jj in range(CK // 16):
            s16 = src_t[ch, pl.ds(jj * 16, 16)]
            d16 = dst_t[ch, pl.ds(jj * 16, 16)]
            e = plsc.load_gather(as_t, [s16]) + plsc.load_gather(ad_t, [d16])
            e = jnp.where(e >= 0, e, 0.2 * e)
            ee = jnp.exp(e)
            for ii in range(16):
                eei = ee[ii]
                r = jj * 16 + ii
                for j in range(DE // 16):
                    buf[p][r, pl.ds(j * 16, 16)] = (
                        buf[p][r, pl.ds(j * 16, 16)] * eei)

    def _chunk(ch, p):
        q = (p + 1) % 3

        @pl.when(ch >= 2)
        def _():
            _swait(ch - 2, q)

        @pl.when(ch + 1 < NCH)
        def _():
            _gather(ch + 1, q)

        _gwait(ch, p)
        _scale(ch, p)
        _scat(ch, p)

    _gather(0, 0)

    def _trip(k, _):
        ch0 = 3 * k
        _chunk(ch0, 0)
        _chunk(ch0 + 1, 1)
        _chunk(ch0 + 2, 2)
        return 0

    lax.fori_loop(0, (NCH - 2) // 3, _trip, 0)
    _chunk(jnp.int32(NCH - 2), 0)
    _chunk(jnp.int32(NCH - 1), 1)
    _swait(jnp.int32(NCH - 2), 0)
    _swait(jnp.int32(NCH - 1), 1)

    plsc.subcore_barrier()

    @pl.when(s < NT)
    def _():
        for j in range(RPT // RC):
            pltpu.sync_copy(acc.at[pl.ds(base + j * RC, RC)], stage)
            pltpu.sync_copy(stage, part_h.at[c, pl.ds(base + j * RC, RC)])


def _edge_phase(src3, dst3, alpha_s, alpha_d, xpe):
    mesh = plsc.VectorSubcoreMesh(
        core_axis_name="c", subcore_axis_name="s",
        num_cores=NC, num_subcores=NS)
    k = pl.kernel(
        _edge_body,
        out_type=jax.ShapeDtypeStruct((NC, N, DE), jnp.float32),
        mesh=mesh,
        compiler_params=pltpu.CompilerParams(
            needs_layout_passes=False, use_tc_tiling_on_sc=False),
        scratch_types=[
            pltpu.VMEM((NCH, CK), jnp.int32),
            pltpu.VMEM((NCH, CK), jnp.int32),
            pltpu.VMEM((N,), jnp.float32),
            pltpu.VMEM((N,), jnp.float32),
            pltpu.VMEM((CK, DE), jnp.float32),
            pltpu.VMEM((CK, DE), jnp.float32),
            pltpu.VMEM((CK, DE), jnp.float32),
            pltpu.VMEM((RC, DE), jnp.float32),
            pltpu.VMEM_SHARED((N, DE), jnp.float32),
        ] + [pltpu.SemaphoreType.DMA] * 6,
    )
    return k(src3, dst3, alpha_s, alpha_d, xpe)


@jax.jit
def kernel(x, edge_index, batch, W0, a_src0, a_dst0, b0,
           W1, a_src1, a_dst1, b1):
    src3 = edge_index[0].reshape(NW, NCH, CK)
    dst3 = edge_index[1].reshape(NW, NCH, CK)
    batchf = batch.astype(jnp.float32).reshape(N, 1)

    xpe0, as0, ad0 = _transform(x, W0, a_src0, a_dst0)
    p0 = _edge_phase(src3, dst3, as0.reshape(N), ad0.reshape(N), xpe0)
    xpe1, as1, ad1 = _mid(p0, b0.reshape(1, 64), W1, a_src1, a_dst1)
    p1 = _edge_phase(src3, dst3, as1.reshape(N), ad1.reshape(N), xpe1)
    return _pool(p1, b1.reshape(1, 64), batchf)


# speedup vs baseline: 1.0913x; 1.0014x over previous
import jax
import jax.numpy as jnp
from jax import lax
from jax.experimental import pallas as pl
from jax.experimental.pallas import tpu as pltpu
from jax.experimental.pallas import tpu_sc as plsc

N = 10000
E = 320000
D_IN = 128
HID = 64
OUT = 64
B = 16

DE = 80
NC = 2
NS = 16
NW = NC * NS
EP = E // NW
CK = 80
NCH = EP // CK
NT = 16
RPT = N // NT
RC = 125
BR = 400
GRID = N // BR

_EPS = 1e-16


def _transform_body(h_ref, w_ref, asrc_ref, adst_ref, xpe_ref, as_ref, ad_ref):
    xp = jnp.dot(h_ref[...], w_ref[...], preferred_element_type=jnp.float32)
    as_ref[...] = jnp.sum(xp * asrc_ref[...], axis=1, keepdims=True)
    ad_ref[...] = jnp.sum(xp * adst_ref[...], axis=1, keepdims=True)
    lane = lax.broadcasted_iota(jnp.int32, (BR, DE - 64), 1)
    pad = jnp.where(lane == 0, 1.0, 0.0).astype(jnp.float32)
    xpe_ref[...] = jnp.concatenate([xp, pad], axis=1)


def _transform(h, w, a_src, a_dst):
    d = h.shape[1]
    return pl.pallas_call(
        _transform_body,
        grid=(GRID,),
        in_specs=[
            pl.BlockSpec((BR, d), lambda i: (i, 0)),
            pl.BlockSpec((d, 64), lambda i: (0, 0)),
            pl.BlockSpec((1, 64), lambda i: (0, 0)),
            pl.BlockSpec((1, 64), lambda i: (0, 0)),
        ],
        out_specs=[
            pl.BlockSpec((BR, DE), lambda i: (i, 0)),
            pl.BlockSpec((BR, 1), lambda i: (i, 0)),
            pl.BlockSpec((BR, 1), lambda i: (i, 0)),
        ],
        out_shape=[
            jax.ShapeDtypeStruct((N, DE), jnp.float32),
            jax.ShapeDtypeStruct((N, 1), jnp.float32),
            jax.ShapeDtypeStruct((N, 1), jnp.float32),
        ],
    )(h, w, a_src, a_dst)


def _mid_body(p_ref, b_ref, w_ref, asrc_ref, adst_ref,
              xpe_ref, as_ref, ad_ref):
    ps = p_ref[0] + p_ref[1]
    den = ps[:, 64:65]
    h = ps[:, :64] / (den + _EPS) + b_ref[...]
    h = jnp.where(h > 0, h, jnp.exp(jnp.minimum(h, 0.0)) - 1.0)
    xp = jnp.dot(h, w_ref[...], preferred_element_type=jnp.float32)
    as_ref[...] = jnp.sum(xp * asrc_ref[...], axis=1, keepdims=True)
    ad_ref[...] = jnp.sum(xp * adst_ref[...], axis=1, keepdims=True)
    lane = lax.broadcasted_iota(jnp.int32, (BR, DE - 64), 1)
    pad = jnp.where(lane == 0, 1.0, 0.0).astype(jnp.float32)
    xpe_ref[...] = jnp.concatenate([xp, pad], axis=1)


def _mid(p, b, w, a_src, a_dst):
    return pl.pallas_call(
        _mid_body,
        grid=(GRID,),
        in_specs=[
            pl.BlockSpec((2, BR, DE), lambda i: (0, i, 0)),
            pl.BlockSpec((1, 64), lambda i: (0, 0)),
            pl.BlockSpec((64, 64), lambda i: (0, 0)),
            pl.BlockSpec((1, 64), lambda i: (0, 0)),
            pl.BlockSpec((1, 64), lambda i: (0, 0)),
        ],
        out_specs=[
            pl.BlockSpec((BR, DE), lambda i: (i, 0)),
            pl.BlockSpec((BR, 1), lambda i: (i, 0)),
            pl.BlockSpec((BR, 1), lambda i: (i, 0)),
        ],
        out_shape=[
            jax.ShapeDtypeStruct((N, DE), jnp.float32),
            jax.ShapeDtypeStruct((N, 1), jnp.float32),
            jax.ShapeDtypeStruct((N, 1), jnp.float32),
        ],
    )(p, b, w, a_src, a_dst)


def _pool_body(p_ref, b_ref, batch_ref, out_ref, cnt_ref):
    i = pl.program_id(0)

    @pl.when(i == 0)
    def _():
        out_ref[...] = jnp.zeros_like(out_ref)
        cnt_ref[...] = jnp.zeros_like(cnt_ref)

    ps = p_ref[0] + p_ref[1]
    den = ps[:, 64:65]
    h = ps[:, :64] / (den + _EPS) + b_ref[...]
    seg = lax.broadcasted_iota(jnp.int32, (BR, B), 1).astype(jnp.float32)
    oh = jnp.where(batch_ref[...] == seg, 1.0, 0.0).astype(jnp.float32)
    out_ref[...] += lax.dot_general(
        oh, h, (((0,), (0,)), ((), ())), preferred_element_type=jnp.float32)
    cnt_ref[...] += jnp.broadcast_to(
        jnp.sum(oh, axis=0, keepdims=True).T, (B, 64))

    @pl.when(i == GRID - 1)
    def _():
        out_ref[...] = out_ref[...] / jnp.maximum(cnt_ref[...], 1.0)


def _pool(p, b, batchf):
    return pl.pallas_call(
        _pool_body,
        grid=(GRID,),
        in_specs=[
            pl.BlockSpec((2, BR, DE), lambda i: (0, i, 0)),
            pl.BlockSpec((1, 64), lambda i: (0, 0)),
            pl.BlockSpec((BR, 1), lambda i: (i, 0)),
        ],
        out_specs=pl.BlockSpec((B, 64), lambda i: (0, 0)),
        out_shape=jax.ShapeDtypeStruct((B, 64), jnp.float32),
        scratch_shapes=[pltpu.VMEM((B, 64), jnp.float32)],
    )(p, b, batchf)


def _edge_body(src_h, dst_h, as_h, ad_h, xpe_h, part_h,
               src_t, dst_t, as_t, ad_t,
               b_0, b_1, b_2, stage, acc,
               gsem0, gsem1, gsem2, ssem0, ssem1, ssem2):
    buf = [b_0, b_1, b_2]
    gsem = [gsem0, gsem1, gsem2]
    ssem = [ssem0, ssem1, ssem2]
    c = lax.axis_index("c")
    s = lax.axis_index("s")
    g = s * NC + c

    def _zrow(r, _):
        for j in range(DE // 16):
            stage[r, pl.ds(j * 16, 16)] = jnp.zeros((16,), jnp.float32)
        return 0
    lax.fori_loop(0, RC, _zrow, 0)
    base = s * RPT

    @pl.when(s < NT)
    def _():
        for j in range(RPT // RC):
            pltpu.sync_copy(stage, acc.at[pl.ds(base + j * RC, RC)])

    pltpu.sync_copy(src_h.at[g], src_t)
    pltpu.sync_copy(dst_h.at[g], dst_t)
    pltpu.sync_copy(as_h, as_t)
    pltpu.sync_copy(ad_h, ad_t)

    plsc.subcore_barrier()

    def _gather(ch, p):
        pltpu.async_copy(xpe_h.at[src_t.at[ch]], buf[p], gsem[p])

    def _gwait(ch, p):
        pltpu.make_async_copy(xpe_h.at[src_t.at[ch]], buf[p], gsem[p]).wait()

    def _scat(ch, p):
        pltpu.async_copy(buf[p], acc.at[dst_t.at[ch]], ssem[p], add=True)

    def _swait(ch, p):
        pltpu.make_async_copy(buf[p], acc.at[dst_t.at[ch]], ssem[p]).wait()

    def _scale(ch, p):
        for jj in range(CK // 16):
            s16 = src_t[ch, pl.ds(jj * 16, 16)]
            d16 = dst_t[ch, pl.ds(jj * 16, 16)]
            e = plsc.load_gather(as_t, [s16]) + plsc.load_gather(ad_t, [d16])
            e = jnp.where(e >= 0, e, 0.2 * e)
            ee = jnp.exp(e)
            for ii in range(16):
                eei = ee[ii]
                r = jj * 16 + ii
                for j in range(DE // 16):
                    buf[p][r, pl.ds(j * 16, 16)] = (
                        buf[p][r, pl.ds(j * 16, 16)] * eei)

    def _chunk(ch, p):
        q = (p + 1) % 3

        @pl.when(ch >= 2)
        def _():
            _swait(ch - 2, q)

        @pl.when(ch + 1 < NCH)
        def _():
            _gather(ch + 1, q)

        _gwait(ch, p)
        _scale(ch, p)
        _scat(ch, p)

    _gather(0, 0)

    def _trip(k, _):
        ch0 = 3 * k
        _chunk(ch0, 0)
        _chunk(ch0 + 1, 1)
        _chunk(ch0 + 2, 2)
        return 0

    lax.fori_loop(0, (NCH - 2) // 3, _trip, 0)
    _chunk(jnp.int32(NCH - 2), 0)
    _chunk(jnp.int32(NCH - 1), 1)
    _swait(jnp.int32(NCH - 2), 0)
    _swait(jnp.int32(NCH - 1), 1)

    plsc.subcore_barrier()

    @pl.when(s < NT)
    def _():
        for j in range(RPT // RC):
            pltpu.sync_copy(acc.at[pl.ds(base + j * RC, RC)], stage)
            pltpu.sync_copy(stage, part_h.at[c, pl.ds(base + j * RC, RC)])


def _edge_phase(src3, dst3, alpha_s, alpha_d, xpe):
    mesh = plsc.VectorSubcoreMesh(
        core_axis_name="c", subcore_axis_name="s",
        num_cores=NC, num_subcores=NS)
    k = pl.kernel(
        _edge_body,
        out_type=jax.ShapeDtypeStruct((NC, N, DE), jnp.float32),
        mesh=mesh,
        compiler_params=pltpu.CompilerParams(
            needs_layout_passes=False, use_tc_tiling_on_sc=False),
        scratch_types=[
            pltpu.VMEM((NCH, CK), jnp.int32),
            pltpu.VMEM((NCH, CK), jnp.int32),
            pltpu.VMEM((N,), jnp.float32),
            pltpu.VMEM((N,), jnp.float32),
            pltpu.VMEM((CK, DE), jnp.float32),
            pltpu.VMEM((CK, DE), jnp.float32),
            pltpu.VMEM((CK, DE), jnp.float32),
            pltpu.VMEM((RC, DE), jnp.float32),
            pltpu.VMEM_SHARED((N, DE), jnp.float32),
        ] + [pltpu.SemaphoreType.DMA] * 6,
    )
    return k(src3, dst3, alpha_s, alpha_d, xpe)


@jax.jit
def kernel(x, edge_index, batch, W0, a_src0, a_dst0, b0,
           W1, a_src1, a_dst1, b1):
    src3 = edge_index[0].reshape(NW, NCH, CK)
    dst3 = edge_index[1].reshape(NW, NCH, CK)
    batchf = batch.astype(jnp.float32).reshape(N, 1)

    xpe0, as0, ad0 = _transform(x, W0, a_src0, a_dst0)
    p0 = _edge_phase(src3, dst3, as0.reshape(N), ad0.reshape(N), xpe0)
    xpe1, as1, ad1 = _mid(p0, b0.reshape(1, 64), W1, a_src1, a_dst1)
    p1 = _edge_phase(src3, dst3, as1.reshape(N), ad1.reshape(N), xpe1)
    return _pool(p1, b1.reshape(1, 64), batchf)
